# baseline (speedup 1.0000x reference)
import jax
import jax.numpy as jnp
from jax.experimental import pallas as pl

PADDING_IDX = 1
SEQ_BLOCK = 256


def _posemb_block(inp_ref, w_ref, out_ref):
    inp = inp_ref[...]
    s, bsz = inp.shape
    dim = w_ref.shape[1]
    mask = (inp != PADDING_IDX).astype(jnp.int32)
    cols = []
    acc = jnp.zeros_like(mask[:, 0:1])
    for b in range(bsz):
        acc = acc + mask[:, b : b + 1]
        cols.append(acc)
    pos = jnp.concatenate(cols, axis=1) * mask + PADDING_IDX
    n_rows = w_ref.shape[0]
    iota = jax.lax.broadcasted_iota(jnp.int32, (1, n_rows), 1)
    w = w_ref[...]
    for b in range(bsz):
        pos_b = pos[:, b : b + 1]
        onehot = (pos_b == iota).astype(jnp.float32)
        out_ref[b, :, :] = jnp.dot(onehot, w, preferred_element_type=jnp.float32)


def kernel(input, weights):
    bsz, seq_len = input.shape
    dim = weights.shape[1]
    n_rows = max(8, bsz + 2)
    grid = (seq_len // SEQ_BLOCK,)
    out = pl.pallas_call(
        _posemb_block,
        grid=grid,
        in_specs=[
            pl.BlockSpec((SEQ_BLOCK, bsz), lambda i: (i, 0)),
            pl.BlockSpec((n_rows, dim), lambda i: (0, 0)),
        ],
        out_specs=pl.BlockSpec((bsz, SEQ_BLOCK, dim), lambda i: (0, i, 0)),
        out_shape=jax.ShapeDtypeStruct((bsz, seq_len, dim), weights.dtype),
    )(input.T, weights)
    return out


# speedup vs baseline: 1.2329x; 1.2329x over previous
import jax
import jax.numpy as jnp
from jax.experimental import pallas as pl

PADDING_IDX = 1
SEQ_BLOCK = 512


def _posemb_block(inp_ref, w_ref, out_ref):
    inp = inp_ref[...].T
    s, bsz = inp.shape
    dim = w_ref.shape[1]
    mask = (inp != PADDING_IDX).astype(jnp.int32)
    cols = []
    acc = jnp.zeros_like(mask[:, 0:1])
    for b in range(bsz):
        acc = acc + mask[:, b : b + 1]
        cols.append(acc)
    pos = jnp.concatenate(cols, axis=1) * mask + PADDING_IDX
    n_rows = w_ref.shape[0]
    iota = jax.lax.broadcasted_iota(jnp.int32, (1, n_rows), 1)
    w = w_ref[...]
    for b in range(bsz):
        pos_b = pos[:, b : b + 1]
        onehot = (pos_b == iota).astype(jnp.float32)
        out_ref[b, :, :] = jnp.dot(onehot, w, preferred_element_type=jnp.float32)


def kernel(input, weights):
    bsz, seq_len = input.shape
    dim = weights.shape[1]
    n_rows = max(8, bsz + 2)
    grid = (seq_len // SEQ_BLOCK,)
    out = pl.pallas_call(
        _posemb_block,
        grid=grid,
        in_specs=[
            pl.BlockSpec((bsz, SEQ_BLOCK), lambda i: (0, i)),
            pl.BlockSpec((n_rows, dim), lambda i: (0, 0)),
        ],
        out_specs=pl.BlockSpec((bsz, SEQ_BLOCK, dim), lambda i: (0, i, 0)),
        out_shape=jax.ShapeDtypeStruct((bsz, seq_len, dim), weights.dtype),
    )(input, weights)
    return out
